# Initial kernel scaffold; baseline (speedup 1.0000x reference)
import jax
import jax.numpy as jnp
from jax.experimental import pallas as pl


def kernel(feat0, feat1, token0, token1, mask_nodes0, keep_nodes0, mask_nodes1, keep_nodes1):
    raise NotImplementedError("write your pallas kernel here")


# speedup vs baseline: 7.9416x; 7.9416x over previous
import functools

import jax
import jax.numpy as jnp
from jax import lax
from jax.experimental import pallas as pl
from jax.experimental.pallas import tpu as pltpu
from jax.experimental.pallas import tpu_sc as plsc

_CHUNK = 128


def _pad_idx(idx):
    n = idx.shape[0]
    pad = (-n) % _CHUNK
    if pad:
        idx = jnp.concatenate([idx, jnp.broadcast_to(idx[:1], (pad,))])
    return idx


def _stream_rows(w, nw, idx_h, idx_v, rows_v, sem, dst_h, src_h=None):
    nchunks = idx_h.shape[0] // _CHUNK
    trips = -(-nchunks // nw)

    def body(i, carry):
        c = w + i * nw

        @pl.when(c < nchunks)
        def _():
            pltpu.sync_copy(idx_h.at[pl.ds(c * _CHUNK, _CHUNK)], idx_v)
            if src_h is not None:
                pltpu.async_copy(src_h.at[idx_v], rows_v, sem).wait()
            pltpu.async_copy(rows_v, dst_h.at[idx_v], sem).wait()

        return carry

    lax.fori_loop(0, trips, body, 0)


def _masker_body(feat0_h, kidx0_h, midx0_h, tok0_h, feat1_h, kidx1_h,
                 midx1_h, tok1_h, out0_h, out1_h, idx_v, rows0_v, rows1_v,
                 sem):
    info = plsc.get_sparse_core_info()
    nc, ns = info.num_cores, info.num_subcores
    nw = nc * ns
    w = lax.axis_index("s") * nc + lax.axis_index("c")

    _stream_rows(w, nw, kidx0_h, idx_v, rows0_v, sem, out0_h, src_h=feat0_h)
    pltpu.sync_copy(tok0_h, rows0_v)
    _stream_rows(w, nw, midx0_h, idx_v, rows0_v, sem, out0_h)

    _stream_rows(w, nw, kidx1_h, idx_v, rows1_v, sem, out1_h, src_h=feat1_h)
    pltpu.sync_copy(tok1_h, rows1_v)
    _stream_rows(w, nw, midx1_h, idx_v, rows1_v, sem, out1_h)


def kernel(feat0, feat1, token0, token1, mask_nodes0, keep_nodes0,
           mask_nodes1, keep_nodes1):
    n0, d0 = feat0.shape
    n1, d1 = feat1.shape
    kidx0 = _pad_idx(keep_nodes0)
    midx0 = _pad_idx(mask_nodes0)
    kidx1 = _pad_idx(keep_nodes1)
    midx1 = _pad_idx(mask_nodes1)
    tok0_rows = jnp.broadcast_to(token0, (_CHUNK, d0))
    tok1_rows = jnp.broadcast_to(token1, (_CHUNK, d1))

    mesh = plsc.VectorSubcoreMesh(core_axis_name="c", subcore_axis_name="s")
    run = functools.partial(
        pl.kernel,
        out_type=(
            jax.ShapeDtypeStruct((n0, d0), feat0.dtype),
            jax.ShapeDtypeStruct((n1, d1), feat1.dtype),
        ),
        mesh=mesh,
        scratch_types=[
            pltpu.VMEM((_CHUNK,), jnp.int32),
            pltpu.VMEM((_CHUNK, d0), jnp.float32),
            pltpu.VMEM((_CHUNK, d1), jnp.float32),
            pltpu.SemaphoreType.DMA,
        ],
    )(_masker_body)
    out0, out1 = run(feat0, kidx0, midx0, tok0_rows, feat1, kidx1, midx1,
                     tok1_rows)
    return (out0, out1, mask_nodes0, keep_nodes0, mask_nodes1, keep_nodes1)


# speedup vs baseline: 10.8740x; 1.3692x over previous
import functools

import jax
import jax.numpy as jnp
from jax import lax
from jax.experimental import pallas as pl
from jax.experimental.pallas import tpu as pltpu
from jax.experimental.pallas import tpu_sc as plsc

_CHUNK = 128
_NW = 32


def _pad_idx(idx):
    n = idx.shape[0]
    npad = -(-n // (_CHUNK * _NW)) * (_CHUNK * _NW)
    if npad != n:
        reps = -(-npad // n)
        idx = jnp.tile(idx, reps)[:npad]
    return idx.reshape(npad // (_CHUNK * _NW), _NW, _CHUNK)


def _keep_phase(w, idx3_h, idx_v, rows, gsems, ssems, src_h, dst_h):
    trips = idx3_h.shape[0]
    pltpu.sync_copy(idx3_h.at[:, w, :], idx_v.at[pl.ds(0, trips), :])
    gather = [None, None]
    scatter = [None, None]
    for i in range(trips):
        p = i % 2
        if scatter[p] is not None:
            scatter[p].wait()
        gather[p] = pltpu.async_copy(src_h.at[idx_v.at[i]], rows[p], gsems[p])
        if i >= 1:
            q = 1 - p
            gather[q].wait()
            scatter[q] = pltpu.async_copy(rows[q], dst_h.at[idx_v.at[i - 1]],
                                          ssems[q])
    p = (trips - 1) % 2
    gather[p].wait()
    scatter[p] = pltpu.async_copy(rows[p], dst_h.at[idx_v.at[trips - 1]],
                                  ssems[p])
    for s in scatter:
        if s is not None:
            s.wait()


def _mask_phase(w, idx3_h, idx_v, tok_rows, ssems, dst_h):
    trips = idx3_h.shape[0]
    pltpu.sync_copy(idx3_h.at[:, w, :], idx_v.at[pl.ds(0, trips), :])
    scatter = [None, None]
    for i in range(trips):
        p = i % 2
        if scatter[p] is not None:
            scatter[p].wait()
        scatter[p] = pltpu.async_copy(tok_rows, dst_h.at[idx_v.at[i]],
                                      ssems[p])
    for s in scatter:
        if s is not None:
            s.wait()


def _masker_body(feat0_h, kidx0_h, midx0_h, tok0_h, feat1_h, kidx1_h,
                 midx1_h, tok1_h, out0_h, out1_h, idx_v, rows0a, rows0b,
                 rows1a, rows1b, gsem0, gsem1, ssem0, ssem1):
    nc = plsc.get_sparse_core_info().num_cores
    w = lax.axis_index("s") * nc + lax.axis_index("c")
    gsems, ssems = (gsem0, gsem1), (ssem0, ssem1)

    _keep_phase(w, kidx0_h, idx_v, (rows0a, rows0b), gsems, ssems,
                feat0_h, out0_h)
    pltpu.sync_copy(tok0_h, rows0a)
    _mask_phase(w, midx0_h, idx_v, rows0a, ssems, out0_h)

    _keep_phase(w, kidx1_h, idx_v, (rows1a, rows1b), gsems, ssems,
                feat1_h, out1_h)
    pltpu.sync_copy(tok1_h, rows1a)
    _mask_phase(w, midx1_h, idx_v, rows1a, ssems, out1_h)


def kernel(feat0, feat1, token0, token1, mask_nodes0, keep_nodes0,
           mask_nodes1, keep_nodes1):
    n0, d0 = feat0.shape
    n1, d1 = feat1.shape
    kidx0 = _pad_idx(keep_nodes0)
    midx0 = _pad_idx(mask_nodes0)
    kidx1 = _pad_idx(keep_nodes1)
    midx1 = _pad_idx(mask_nodes1)
    tok0_rows = jnp.broadcast_to(token0, (_CHUNK, d0))
    tok1_rows = jnp.broadcast_to(token1, (_CHUNK, d1))
    max_trips = max(kidx0.shape[0], midx0.shape[0], kidx1.shape[0],
                    midx1.shape[0])

    mesh = plsc.VectorSubcoreMesh(core_axis_name="c", subcore_axis_name="s")
    run = functools.partial(
        pl.kernel,
        out_type=(
            jax.ShapeDtypeStruct((n0, d0), feat0.dtype),
            jax.ShapeDtypeStruct((n1, d1), feat1.dtype),
        ),
        mesh=mesh,
        scratch_types=[
            pltpu.VMEM((max_trips, _CHUNK), jnp.int32),
            pltpu.VMEM((_CHUNK, d0), jnp.float32),
            pltpu.VMEM((_CHUNK, d0), jnp.float32),
            pltpu.VMEM((_CHUNK, d1), jnp.float32),
            pltpu.VMEM((_CHUNK, d1), jnp.float32),
            pltpu.SemaphoreType.DMA,
            pltpu.SemaphoreType.DMA,
            pltpu.SemaphoreType.DMA,
            pltpu.SemaphoreType.DMA,
        ],
    )(_masker_body)
    out0, out1 = run(feat0, kidx0, midx0, tok0_rows, feat1, kidx1, midx1,
                     tok1_rows)
    return (out0, out1, mask_nodes0, keep_nodes0, mask_nodes1, keep_nodes1)


# speedup vs baseline: 10.9714x; 1.0090x over previous
import functools

import jax
import jax.numpy as jnp
from jax import lax
from jax.experimental import pallas as pl
from jax.experimental.pallas import tpu as pltpu
from jax.experimental.pallas import tpu_sc as plsc

_CHUNK = 128
_NW = 32


def _pad_idx(idx):
    n = idx.shape[0]
    npad = -(-n // (_CHUNK * _NW)) * (_CHUNK * _NW)
    if npad != n:
        reps = -(-npad // n)
        idx = jnp.tile(idx, reps)[:npad]
    return idx.reshape(npad // (_CHUNK * _NW), _NW, _CHUNK)


class _KeepStream:
    def __init__(self, w, idx3_h, idx_v, rows, gsems, ssems, src_h, dst_h):
        self.trips = idx3_h.shape[0]
        self.idx_v, self.rows = idx_v, rows
        self.gsems, self.ssems = gsems, ssems
        self.src_h, self.dst_h = src_h, dst_h
        self.idx_dma = pltpu.async_copy(idx3_h.at[:, w, :], idx_v, gsems[0])
        self.gather = [None, None]
        self.scatter = [None, None]

    def step(self, i):
        p = i % 2
        if i == 0:
            self.idx_dma.wait()
        if self.scatter[p] is not None:
            self.scatter[p].wait()
        self.gather[p] = pltpu.async_copy(
            self.src_h.at[self.idx_v.at[i]], self.rows[p], self.gsems[p])
        if i >= 1:
            q = 1 - p
            self.gather[q].wait()
            self.scatter[q] = pltpu.async_copy(
                self.rows[q], self.dst_h.at[self.idx_v.at[i - 1]],
                self.ssems[q])

    def drain(self):
        p = (self.trips - 1) % 2
        self.gather[p].wait()
        self.scatter[p] = pltpu.async_copy(
            self.rows[p], self.dst_h.at[self.idx_v.at[self.trips - 1]],
            self.ssems[p])
        for s in self.scatter:
            if s is not None:
                s.wait()


class _MaskStream:
    def __init__(self, w, idx3_h, idx_v, tok_h, tok_rows, ssems):
        self.trips = idx3_h.shape[0]
        self.idx_v, self.tok_rows, self.ssems = idx_v, tok_rows, ssems
        self.idx_dma = pltpu.async_copy(idx3_h.at[:, w, :], idx_v, ssems[0])
        self.tok_h = tok_h
        self.scatter = [None, None]
        self.dst_h = None

    def start(self, dst_h):
        self.dst_h = dst_h
        self.idx_dma.wait()
        pltpu.sync_copy(self.tok_h, self.tok_rows)

    def step(self, i):
        p = i % 2
        if self.scatter[p] is not None:
            self.scatter[p].wait()
        self.scatter[p] = pltpu.async_copy(
            self.tok_rows, self.dst_h.at[self.idx_v.at[i]], self.ssems[p])

    def drain(self):
        for s in self.scatter:
            if s is not None:
                s.wait()


def _masker_body(feat0_h, kidx0_h, midx0_h, tok0_h, feat1_h, kidx1_h,
                 midx1_h, tok1_h, out0_h, out1_h, kv0, kv1, mv0, mv1,
                 rows0a, rows0b, rows1a, rows1b,
                 g0a, g0b, s0a, s0b, g1a, g1b, s1a, s1b, m0a, m0b, m1a, m1b):
    nc = plsc.get_sparse_core_info().num_cores
    w = lax.axis_index("s") * nc + lax.axis_index("c")

    keep0 = _KeepStream(w, kidx0_h, kv0, (rows0a, rows0b), (g0a, g0b),
                        (s0a, s0b), feat0_h, out0_h)
    keep1 = _KeepStream(w, kidx1_h, kv1, (rows1a, rows1b), (g1a, g1b),
                        (s1a, s1b), feat1_h, out1_h)
    mask0 = _MaskStream(w, midx0_h, mv0, tok0_h, rows0a, (m0a, m0b))
    mask1 = _MaskStream(w, midx1_h, mv1, tok1_h, rows1a, (m1a, m1b))

    t0, t1 = keep0.trips, keep1.trips
    for i in range(t0):
        keep0.step(i)
        if i < t1:
            keep1.step(i)
        if i == t1:
            keep1.drain()
            mask1.tok_rows = rows1a
            mask1.start(out1_h)
        j = i - t1 - 1
        if 0 <= j < mask1.trips:
            mask1.step(j)
    keep0.drain()
    mask1.drain()
    mask0.start(out0_h)
    for i in range(mask0.trips):
        mask0.step(i)
    mask0.drain()


def kernel(feat0, feat1, token0, token1, mask_nodes0, keep_nodes0,
           mask_nodes1, keep_nodes1):
    n0, d0 = feat0.shape
    n1, d1 = feat1.shape
    kidx0 = _pad_idx(keep_nodes0)
    midx0 = _pad_idx(mask_nodes0)
    kidx1 = _pad_idx(keep_nodes1)
    midx1 = _pad_idx(mask_nodes1)
    tok0_rows = jnp.broadcast_to(token0, (_CHUNK, d0))
    tok1_rows = jnp.broadcast_to(token1, (_CHUNK, d1))

    mesh = plsc.VectorSubcoreMesh(core_axis_name="c", subcore_axis_name="s")
    dma = pltpu.SemaphoreType.DMA
    run = functools.partial(
        pl.kernel,
        out_type=(
            jax.ShapeDtypeStruct((n0, d0), feat0.dtype),
            jax.ShapeDtypeStruct((n1, d1), feat1.dtype),
        ),
        mesh=mesh,
        scratch_types=[
            pltpu.VMEM((kidx0.shape[0], _CHUNK), jnp.int32),
            pltpu.VMEM((kidx1.shape[0], _CHUNK), jnp.int32),
            pltpu.VMEM((midx0.shape[0], _CHUNK), jnp.int32),
            pltpu.VMEM((midx1.shape[0], _CHUNK), jnp.int32),
            pltpu.VMEM((_CHUNK, d0), jnp.float32),
            pltpu.VMEM((_CHUNK, d0), jnp.float32),
            pltpu.VMEM((_CHUNK, d1), jnp.float32),
            pltpu.VMEM((_CHUNK, d1), jnp.float32),
        ] + [dma] * 12,
    )(_masker_body)
    out0, out1 = run(feat0, kidx0, midx0, tok0_rows, feat1, kidx1, midx1,
                     tok1_rows)
    return (out0, out1, mask_nodes0, keep_nodes0, mask_nodes1, keep_nodes1)
